# baseline (speedup 1.0000x reference)
import functools

import jax
import jax.numpy as jnp
from jax import lax
from jax.experimental import pallas as pl
from jax.experimental.pallas import tpu as pltpu
from jax.experimental.pallas import tpu_sc as plsc

N_ATOMS = 320000
NUM_BLOCKS = 32000
NUM_GRAPHS = 64
HIDDEN = 128
N_LEVELS = 50

ATILE = 1280
NTILES = N_ATOMS // ATILE

BCHUNK = 3200
NBSTEPS = NUM_BLOCKS // BCHUNK

_NC, _NS = 2, 16
_NW = _NC * _NS
_CH = 1000
_GSTEPS = N_ATOMS // (_NW * _CH)
_RPS = NUM_BLOCKS // _NS


def _silu(x):
    return x * jax.nn.sigmoid(x)


def _bf(x):
    return x.astype(jnp.bfloat16)


def _iota16():
    return lax.broadcasted_iota(jnp.int32, (16,), 0)


def _sc_gather_body(tablet_hbm, idx_hbm, out_hbm, idx_v, plane_v, sem):
    wid = lax.axis_index("s") * _NC + lax.axis_index("c")
    base = wid * (_CH * _GSTEPS)

    def step(j, carry):
        off = base + j * _CH
        pltpu.sync_copy(idx_hbm.at[pl.ds(off, _CH)], idx_v)
        for f in range(4):
            pltpu.async_copy(tablet_hbm.at[f].at[idx_v], plane_v, sem).wait()
            pltpu.sync_copy(plane_v, out_hbm.at[f, pl.ds(off, _CH)])
        return carry

    lax.fori_loop(0, _GSTEPS, step, 0)


def _sc_gather(tablet, idx):
    return pl.kernel(
        _sc_gather_body,
        out_type=jax.ShapeDtypeStruct((4, N_ATOMS), jnp.float32),
        mesh=plsc.VectorSubcoreMesh(core_axis_name="c", subcore_axis_name="s",
                                    num_cores=_NC, num_subcores=_NS),
        scratch_types=[
            pltpu.VMEM((_CH,), jnp.int32),
            pltpu.VMEM((_CH,), jnp.float32),
            pltpu.SemaphoreType.DMA,
        ],
        compiler_params=pltpu.CompilerParams(use_tc_tiling_on_sc=False,
                                             needs_layout_passes=False),
    )(tablet, idx)


def _sc_scatter_add(predt, idx, zeros, ones_atoms):
    def body(predt_hbm, idx_hbm, zeros_hbm, ones_hbm, out_hbm, idx_v,
             plane_v, one_v, acc_sh, sem):
        cid = lax.axis_index("c")
        sid = lax.axis_index("s")

        @pl.when(sid == 0)
        def _():
            pltpu.sync_copy(zeros_hbm, acc_sh)

        base = (cid * _NS + sid) * (_CH * _GSTEPS)
        pltpu.sync_copy(ones_hbm.at[pl.ds(0, _CH)], one_v)
        plsc.subcore_barrier()

        def step(j, carry):
            off = base + j * _CH
            pltpu.sync_copy(idx_hbm.at[pl.ds(off, _CH)], idx_v)
            for f in range(3):
                pltpu.sync_copy(predt_hbm.at[f, pl.ds(off, _CH)], plane_v)
                pltpu.sync_copy(plane_v, acc_sh.at[f].at[idx_v], add=True)
            pltpu.sync_copy(one_v, acc_sh.at[3].at[idx_v], add=True)
            return carry

        lax.fori_loop(0, _GSTEPS, step, 0)
        plsc.subcore_barrier()
        for f in range(4):
            pltpu.sync_copy(acc_sh.at[f, pl.ds(sid * _RPS, _RPS)],
                            out_hbm.at[cid, f, pl.ds(sid * _RPS, _RPS)])

    return pl.kernel(
        body,
        out_type=jax.ShapeDtypeStruct((_NC, 4, NUM_BLOCKS), jnp.float32),
        mesh=plsc.VectorSubcoreMesh(core_axis_name="c", subcore_axis_name="s",
                                    num_cores=_NC, num_subcores=_NS),
        scratch_types=[
            pltpu.VMEM((_CH,), jnp.int32),
            pltpu.VMEM((_CH,), jnp.float32),
            pltpu.VMEM((_CH,), jnp.float32),
            pltpu.VMEM_SHARED((4, NUM_BLOCKS), jnp.float32),
            pltpu.SemaphoreType.DMA,
        ],
        compiler_params=pltpu.CompilerParams(use_tc_tiling_on_sc=False,
                                             needs_layout_passes=False),
    )(predt, idx, zeros, ones_atoms)


def _sc_hist(block_idx, batch_idx, zeros, ones_atoms):
    def body(bid_hbm, gid_hbm, zeros_hbm, ones_hbm, out_hbm, idx_v, one_v,
             acc_sh, sem):
        cid = lax.axis_index("c")
        sid = lax.axis_index("s")

        @pl.when(sid == 0)
        def _():
            pltpu.sync_copy(zeros_hbm, acc_sh)

        wid = cid * _NS + sid
        base = wid * (_CH * _GSTEPS)
        pltpu.sync_copy(ones_hbm.at[pl.ds(0, _CH)], one_v)
        plsc.subcore_barrier()

        def step(j, carry):
            off = base + j * _CH
            pltpu.sync_copy(bid_hbm.at[pl.ds(off, _CH)], idx_v)
            pltpu.sync_copy(one_v, acc_sh.at[0].at[idx_v], add=True)
            return carry

        lax.fori_loop(0, _GSTEPS, step, 0)
        pltpu.sync_copy(gid_hbm.at[pl.ds(wid * _CH, _CH)], idx_v)
        pltpu.sync_copy(one_v, acc_sh.at[1].at[idx_v], add=True)
        plsc.subcore_barrier()
        for p in range(2):
            pltpu.sync_copy(acc_sh.at[p, pl.ds(sid * _RPS, _RPS)],
                            out_hbm.at[cid, p, pl.ds(sid * _RPS, _RPS)])

    return pl.kernel(
        body,
        out_type=jax.ShapeDtypeStruct((_NC, 2, NUM_BLOCKS), jnp.float32),
        mesh=plsc.VectorSubcoreMesh(core_axis_name="c", subcore_axis_name="s",
                                    num_cores=_NC, num_subcores=_NS),
        scratch_types=[
            pltpu.VMEM((_CH,), jnp.int32),
            pltpu.VMEM((_CH,), jnp.float32),
            pltpu.VMEM_SHARED((2, NUM_BLOCKS), jnp.float32),
            pltpu.SemaphoreType.DMA,
        ],
        compiler_params=pltpu.CompilerParams(use_tc_tiling_on_sc=False,
                                             needs_layout_passes=False),
    )(block_idx, batch_idx, zeros, ones_atoms)


PCHUNK = 3200
NPSTEPS = NUM_BLOCKS // PCHUNK


def _prep_body(c_ref, nb64_ref, nl_ref, sig_ref, brow_ref, bcol_ref,
               noiset_ref, tablet_ref, apg_ref):
    i = pl.program_id(0)

    @pl.when(i == 0)
    def _():
        apg_ref[...] = jnp.zeros_like(apg_ref)

    nl_row = nl_ref[...]
    lev_col = lax.broadcasted_iota(jnp.int32, (N_LEVELS, NUM_GRAPHS), 0)
    onehot_nl = (lev_col == nl_row).astype(jnp.float32)
    sig_g = jnp.dot(sig_ref[...], onehot_nl,
                    preferred_element_type=jnp.float32)
    brow = brow_ref[...]
    gcol = lax.broadcasted_iota(jnp.int32, (NUM_GRAPHS, PCHUNK), 0)
    og = (gcol == brow).astype(jnp.float32)
    sig_b = jnp.dot(sig_g, og, preferred_element_type=jnp.float32)
    nb_b = jnp.dot(nb64_ref[...], og, preferred_element_type=jnp.float32)
    c_row = c_ref[...]
    wb = 1.0 / (jnp.maximum(c_row, 1.0) * jnp.maximum(nb_b, 1.0))
    tablet_ref[...] = jnp.concatenate(
        [noiset_ref[...] * sig_b, wb], axis=0)
    bcol = bcol_ref[...]
    grow = lax.broadcasted_iota(jnp.int32, (PCHUNK, NUM_GRAPHS), 1)
    ogt = (grow == bcol).astype(jnp.float32)
    apg_ref[...] += jnp.dot(c_row, ogt, preferred_element_type=jnp.float32)


def _prep(c_row, nb64, nl_row, sig_row, brow, bcol, noiset):
    return pl.pallas_call(
        _prep_body,
        grid=(NPSTEPS,),
        in_specs=[
            pl.BlockSpec((1, PCHUNK), lambda i: (0, i)),
            pl.BlockSpec((1, NUM_GRAPHS), lambda i: (0, 0)),
            pl.BlockSpec((1, NUM_GRAPHS), lambda i: (0, 0)),
            pl.BlockSpec((1, N_LEVELS), lambda i: (0, 0)),
            pl.BlockSpec((1, PCHUNK), lambda i: (0, i)),
            pl.BlockSpec((PCHUNK, 1), lambda i: (i, 0)),
            pl.BlockSpec((3, PCHUNK), lambda i: (0, i)),
        ],
        out_specs=[
            pl.BlockSpec((4, PCHUNK), lambda i: (0, i)),
            pl.BlockSpec((1, NUM_GRAPHS), lambda i: (0, 0)),
        ],
        out_shape=[
            jax.ShapeDtypeStruct((4, NUM_BLOCKS), jnp.float32),
            jax.ShapeDtypeStruct((1, NUM_GRAPHS), jnp.float32),
        ],
    )(c_row, nb64, nl_row, sig_row, brow, bcol, noiset)


def _fused_body(z_ref, g_ref, abound_ref, h_ref, wenc_ref, wpos_ref,
                woutt_ref, predt_ref, gacc_ref):
    i = pl.program_id(0)

    @pl.when(i == 0)
    def _():
        gacc_ref[...] = jnp.zeros_like(gacc_ref)

    g = g_ref[...]
    x = jnp.dot(_bf(h_ref[...]), _bf(wenc_ref[...]),
                preferred_element_type=jnp.float32)
    x = x + jnp.dot(z_ref[...], wpos_ref[...],
                    preferred_element_type=jnp.float32)
    x = x + lax.dot_general(g[0:3, :], wpos_ref[...],
                            dimension_numbers=(((0,), (0,)), ((), ())),
                            preferred_element_type=jnp.float32)
    u = _silu(x)
    predt_ref[...] = lax.dot_general(
        woutt_ref[...], u,
        dimension_numbers=(((1,), (1,)), ((), ())),
        preferred_element_type=jnp.float32)
    aidx = (i * ATILE
            + lax.broadcasted_iota(jnp.int32, (NUM_GRAPHS, ATILE), 1))
    bound = abound_ref[...]
    onehot_t = ((aidx >= bound[0:NUM_GRAPHS, :])
                & (aidx < bound[1:NUM_GRAPHS + 1, :])).astype(jnp.float32)
    owt = onehot_t * g[3:4, :]
    gacc_ref[...] += jnp.dot(_bf(owt), _bf(u),
                             preferred_element_type=jnp.float32)


def _finalize_body(sp0_ref, sp1_ref, noiset_ref, gacc_ref, w1_ref, b1_ref,
                   w2_ref, b2_ref, energy_ref, loss_ref):
    i = pl.program_id(0)

    @pl.when(i == 0)
    def _():
        loss_ref[...] = jnp.zeros_like(loss_ref)

    sp = sp0_ref[...] + sp1_ref[...]
    c = sp[3:4, :]
    m = (c > 0.0).astype(jnp.float32)
    d = sp[0:3, :] / jnp.maximum(c, 1.0) + noiset_ref[...] * m
    loss_ref[...] += (jnp.sum(d * d) / (NUM_BLOCKS * 3.0)).reshape(1, 1)

    @pl.when(i == NBSTEPS - 1)
    def _():
        hg = _silu(jnp.dot(gacc_ref[...], w1_ref[...],
                           preferred_element_type=jnp.float32) + b1_ref[...])
        energy_ref[...] = jnp.dot(hg, w2_ref[...],
                                  preferred_element_type=jnp.float32) \
            + b2_ref[...]


@jax.jit
def kernel(Z, H, noise, sigmas, W_enc, W_pos, W_out, W1, b1, W2, b2,
           block_id, batch_id, noise_level):
    f32 = jnp.float32
    hist = _sc_hist(block_id, batch_id, jnp.zeros((2, NUM_BLOCKS), f32),
                    jnp.ones((_CH,), f32))
    c_row = (hist[0, 0] + hist[1, 0]).reshape(1, NUM_BLOCKS)
    nb64 = (hist[0, 1, :NUM_GRAPHS]
            + hist[1, 1, :NUM_GRAPHS]).reshape(1, NUM_GRAPHS)
    noiset = noise.T
    tablet, apg = _prep(c_row, nb64, noise_level.reshape(1, NUM_GRAPHS),
                        sigmas.reshape(1, N_LEVELS),
                        batch_id.reshape(1, NUM_BLOCKS),
                        batch_id.reshape(NUM_BLOCKS, 1), noiset)
    abound = jnp.concatenate(
        [jnp.zeros((1,), f32), jnp.cumsum(apg[0])]).astype(
            jnp.int32).reshape(NUM_GRAPHS + 1, 1)

    g4 = _sc_gather(tablet, block_id)

    predt, graph_repr = pl.pallas_call(
        _fused_body,
        grid=(NTILES,),
        in_specs=[
            pl.BlockSpec((ATILE, 3), lambda i: (i, 0)),
            pl.BlockSpec((4, ATILE), lambda i: (0, i)),
            pl.BlockSpec((NUM_GRAPHS + 1, 1), lambda i: (0, 0)),
            pl.BlockSpec((ATILE, HIDDEN), lambda i: (i, 0)),
            pl.BlockSpec((HIDDEN, HIDDEN), lambda i: (0, 0)),
            pl.BlockSpec((3, HIDDEN), lambda i: (0, 0)),
            pl.BlockSpec((3, HIDDEN), lambda i: (0, 0)),
        ],
        out_specs=[
            pl.BlockSpec((3, ATILE), lambda i: (0, i)),
            pl.BlockSpec((NUM_GRAPHS, HIDDEN), lambda i: (0, 0)),
        ],
        out_shape=[
            jax.ShapeDtypeStruct((3, N_ATOMS), f32),
            jax.ShapeDtypeStruct((NUM_GRAPHS, HIDDEN), f32),
        ],
    )(Z, g4, abound, H, W_enc, W_pos, W_out.T)

    sp = _sc_scatter_add(predt, block_id, jnp.zeros((4, NUM_BLOCKS), f32),
                         jnp.ones((_CH,), f32))

    energy2, loss2 = pl.pallas_call(
        _finalize_body,
        grid=(NBSTEPS,),
        in_specs=[
            pl.BlockSpec((4, BCHUNK), lambda i: (0, i)),
            pl.BlockSpec((4, BCHUNK), lambda i: (0, i)),
            pl.BlockSpec((3, BCHUNK), lambda i: (0, i)),
            pl.BlockSpec((NUM_GRAPHS, HIDDEN), lambda i: (0, 0)),
            pl.BlockSpec((HIDDEN, HIDDEN), lambda i: (0, 0)),
            pl.BlockSpec((1, HIDDEN), lambda i: (0, 0)),
            pl.BlockSpec((HIDDEN, 1), lambda i: (0, 0)),
            pl.BlockSpec((1, 1), lambda i: (0, 0)),
        ],
        out_specs=[
            pl.BlockSpec((NUM_GRAPHS, 1), lambda i: (0, 0)),
            pl.BlockSpec((1, 1), lambda i: (0, 0)),
        ],
        out_shape=[
            jax.ShapeDtypeStruct((NUM_GRAPHS, 1), f32),
            jax.ShapeDtypeStruct((1, 1), f32),
        ],
    )(sp[0], sp[1], noiset, graph_repr, W1, b1[None, :], W2, b2[None, :])

    return energy2[:, 0], graph_repr, loss2[0, 0]
